# baseline (speedup 1.0000x reference)
import jax
import jax.numpy as jnp
from jax.experimental import pallas as pl
from jax.experimental.pallas import tpu as pltpu


def _ln_mm1_body(x_ref, mu_ref, var_ref, g_ref, b_ref, w1_ref, b1_ref, o_ref):
    x = x_ref[0]
    xn = (x - mu_ref[0]) / jnp.sqrt(var_ref[0] + 1e-5) * g_ref[0] + b_ref[0]
    o_ref[0] = xn @ w1_ref[...] + b1_ref[0]


def _mm2_body(h_ref, gm_ref, w2_ref, b2_ref, o_ref):
    h = h_ref[0]
    n, c = h.shape
    x2 = jnp.concatenate(
        [h[:, : c // 2], jnp.broadcast_to(gm_ref[0], (n, c // 2))], axis=-1)
    o_ref[0] = x2 @ w2_ref[...] + b2_ref[0]


_SORT_CHUNK = 128


def _mm3_sort_body(h2_ref, w3_ref, o_ref, score_ref, rank_ref):
    p = pl.program_id(1)
    c = pl.program_id(2)
    ch = _SORT_CHUNK
    n = score_ref.shape[0]

    @pl.when(p == 0)
    def _score():
        logit = h2_ref[0] @ w3_ref[...]
        score_ref[pl.ds(c * ch, ch), :] = jax.nn.sigmoid(logit)

    @pl.when(p == 1)
    def _rank():
        scol = score_ref[...]
        schunk = score_ref[pl.ds(c * ch, ch), :].reshape(1, ch)
        jjc = jax.lax.broadcasted_iota(jnp.int32, (n, ch), 0)
        iic = jax.lax.broadcasted_iota(jnp.int32, (n, ch), 1) + c * ch
        cmp = (scol > schunk) | ((scol == schunk) & (jjc < iic))
        rank_ref[0, pl.ds(c * ch, ch)] = jnp.sum(cmp.astype(jnp.int32), axis=0)

    @pl.when(p == 2)
    def _invert():
        rcol = rank_ref[0, :].reshape(n, 1)
        rr = jax.lax.broadcasted_iota(jnp.int32, (n, ch), 1) + c * ch
        jjc = jax.lax.broadcasted_iota(jnp.int32, (n, ch), 0)
        o_ref[0, 0, :] = jnp.sum(jnp.where(rcol == rr, jjc, 0), axis=0)


def _zmap2(b):
    return (0, 0)


def _bmap3(b):
    return (b, 0, 0)


def kernel(input_x, quality, ln_g, ln_b, w1, b1, w2, b2, w3):
    B, H, W, D = input_x.shape
    N = H * W
    C1 = w1.shape[1]
    C2 = w2.shape[1]
    f32 = jnp.float32

    mu = jnp.mean(input_x, axis=-1, keepdims=True)
    var = jnp.mean((input_x - mu) ** 2, axis=-1, keepdims=True)

    y1 = pl.pallas_call(
        _ln_mm1_body,
        grid=(B,),
        in_specs=[
            pl.BlockSpec((1, N, D), _bmap3),
            pl.BlockSpec((1, N, 1), _bmap3),
            pl.BlockSpec((1, N, 1), _bmap3),
            pl.BlockSpec((1, D), _zmap2),
            pl.BlockSpec((1, D), _zmap2),
            pl.BlockSpec((D, C1), _zmap2),
            pl.BlockSpec((1, C1), _zmap2),
        ],
        out_specs=pl.BlockSpec((1, N, C1), _bmap3),
        out_shape=jax.ShapeDtypeStruct((B, N, C1), f32),
    )(input_x.reshape(B, N, D), mu.reshape(B, N, 1), var.reshape(B, N, 1),
      ln_g.reshape(1, D), ln_b.reshape(1, D), w1, b1.reshape(1, C1))

    h = jax.nn.gelu(y1.reshape(B, H, W, C1), approximate=False)
    gmean = jnp.mean(h[:, :, :, C1 // 2:], axis=(1, 2), keepdims=True)

    y2 = pl.pallas_call(
        _mm2_body,
        grid=(B,),
        in_specs=[
            pl.BlockSpec((1, N, C1), _bmap3),
            pl.BlockSpec((1, 1, C1 // 2), _bmap3),
            pl.BlockSpec((C1, C2), _zmap2),
            pl.BlockSpec((1, C2), _zmap2),
        ],
        out_specs=pl.BlockSpec((1, N, C2), _bmap3),
        out_shape=jax.ShapeDtypeStruct((B, N, C2), f32),
    )(h.reshape(B, N, C1), gmean.reshape(B, 1, C1 // 2), w2,
      b2.reshape(1, C2))

    h2 = jax.nn.gelu(y2, approximate=False)

    ch = _SORT_CHUNK
    perm = pl.pallas_call(
        _mm3_sort_body,
        grid=(B, 3, N // ch),
        in_specs=[
            pl.BlockSpec((1, ch, C2), lambda b, p, c: (b, c, 0)),
            pl.BlockSpec((C2, 1), lambda b, p, c: (0, 0)),
        ],
        out_specs=pl.BlockSpec((1, 1, ch), lambda b, p, c: (b, 0, c)),
        out_shape=jax.ShapeDtypeStruct((B, 1, N), jnp.int32),
        scratch_shapes=[pltpu.VMEM((N, 1), f32), pltpu.VMEM((1, N), jnp.int32)],
    )(h2, w3)

    idx = perm.reshape(B, N)
    log_base = 5.0
    quality_static = 4
    ratio = (log_base ** ((quality_static - 1) / 7.0) - 1.0) / (log_base - 1.0)
    num_keep = int(N * ratio)
    return (input_x, idx[:, :num_keep], idx[:, num_keep:])


# speedup vs baseline: 2.0917x; 2.0917x over previous
import jax
import jax.numpy as jnp
from jax.experimental import pallas as pl
from jax.experimental.pallas import tpu as pltpu


def _ln_mm1_body(x_ref, mu_ref, var_ref, g_ref, b_ref, w1_ref, b1_ref, o_ref):
    x = x_ref[0]
    xn = (x - mu_ref[0]) / jnp.sqrt(var_ref[0] + 1e-5) * g_ref[0] + b_ref[0]
    o_ref[0] = xn @ w1_ref[...] + b1_ref[0]


def _mm2_body(h_ref, gm_ref, w2_ref, b2_ref, o_ref):
    h = h_ref[0]
    n, c = h.shape
    x2 = jnp.concatenate(
        [h[:, : c // 2], jnp.broadcast_to(gm_ref[0], (n, c // 2))], axis=-1)
    o_ref[0] = x2 @ w2_ref[...] + b2_ref[0]


def _mm3_score_body(h2_ref, w3_ref, o_ref):
    logit = h2_ref[0] @ w3_ref[...]
    n = logit.shape[0]
    o_ref[0, 0] = jax.nn.sigmoid(logit).reshape(n)


def _bitonic_body(s_ref, o_ref):
    keys = jax.lax.bitcast_convert_type(s_ref[...], jnp.int32)
    b, n = keys.shape
    idx = jax.lax.broadcasted_iota(jnp.int32, (b, n), 1)
    lane = jax.lax.broadcasted_iota(jnp.int32, (1, n), 1)

    def exchange(keys, idx, j, k):
        km = jnp.concatenate([keys[:, j:], keys[:, :j]], axis=1)
        kp = jnp.concatenate([keys[:, n - j:], keys[:, :n - j]], axis=1)
        im = jnp.concatenate([idx[:, j:], idx[:, :j]], axis=1)
        ip = jnp.concatenate([idx[:, n - j:], idx[:, :n - j]], axis=1)
        is_low = (lane & j) == 0
        pk = jnp.where(is_low, km, kp)
        pi = jnp.where(is_low, im, ip)
        less = (keys > pk) | ((keys == pk) & (idx < pi))
        asc = (lane & k) == 0
        keep = less ^ (~is_low) ^ (~asc)
        return jnp.where(keep, keys, pk), jnp.where(keep, idx, pi)

    k = 2
    while k <= n:
        j = k // 2
        while j >= 1:
            keys, idx = exchange(keys, idx, j, k)
            j //= 2
        k *= 2
    o_ref[...] = idx


def _zmap2(b):
    return (0, 0)


def _bmap3(b):
    return (b, 0, 0)


def kernel(input_x, quality, ln_g, ln_b, w1, b1, w2, b2, w3):
    B, H, W, D = input_x.shape
    N = H * W
    C1 = w1.shape[1]
    C2 = w2.shape[1]
    f32 = jnp.float32

    mu = jnp.mean(input_x, axis=-1, keepdims=True)
    var = jnp.mean((input_x - mu) ** 2, axis=-1, keepdims=True)

    y1 = pl.pallas_call(
        _ln_mm1_body,
        grid=(B,),
        in_specs=[
            pl.BlockSpec((1, N, D), _bmap3),
            pl.BlockSpec((1, N, 1), _bmap3),
            pl.BlockSpec((1, N, 1), _bmap3),
            pl.BlockSpec((1, D), _zmap2),
            pl.BlockSpec((1, D), _zmap2),
            pl.BlockSpec((D, C1), _zmap2),
            pl.BlockSpec((1, C1), _zmap2),
        ],
        out_specs=pl.BlockSpec((1, N, C1), _bmap3),
        out_shape=jax.ShapeDtypeStruct((B, N, C1), f32),
    )(input_x.reshape(B, N, D), mu.reshape(B, N, 1), var.reshape(B, N, 1),
      ln_g.reshape(1, D), ln_b.reshape(1, D), w1, b1.reshape(1, C1))

    h = jax.nn.gelu(y1.reshape(B, H, W, C1), approximate=False)
    gmean = jnp.mean(h[:, :, :, C1 // 2:], axis=(1, 2), keepdims=True)

    y2 = pl.pallas_call(
        _mm2_body,
        grid=(B,),
        in_specs=[
            pl.BlockSpec((1, N, C1), _bmap3),
            pl.BlockSpec((1, 1, C1 // 2), _bmap3),
            pl.BlockSpec((C1, C2), _zmap2),
            pl.BlockSpec((1, C2), _zmap2),
        ],
        out_specs=pl.BlockSpec((1, N, C2), _bmap3),
        out_shape=jax.ShapeDtypeStruct((B, N, C2), f32),
    )(h.reshape(B, N, C1), gmean.reshape(B, 1, C1 // 2), w2,
      b2.reshape(1, C2))

    h2 = jax.nn.gelu(y2, approximate=False)

    scores = pl.pallas_call(
        _mm3_score_body,
        grid=(B,),
        in_specs=[
            pl.BlockSpec((1, N, C2), _bmap3),
            pl.BlockSpec((C2, 1), _zmap2),
        ],
        out_specs=pl.BlockSpec((1, 1, N), _bmap3),
        out_shape=jax.ShapeDtypeStruct((B, 1, N), f32),
    )(h2, w3)

    idx = pl.pallas_call(
        _bitonic_body,
        in_specs=[pl.BlockSpec((B, N), lambda: (0, 0))],
        out_specs=pl.BlockSpec((B, N), lambda: (0, 0)),
        out_shape=jax.ShapeDtypeStruct((B, N), jnp.int32),
    )(scores.reshape(B, N))
    log_base = 5.0
    quality_static = 4
    ratio = (log_base ** ((quality_static - 1) / 7.0) - 1.0) / (log_base - 1.0)
    num_keep = int(N * ratio)
    return (input_x, idx[:, :num_keep], idx[:, num_keep:])


# speedup vs baseline: 2.5094x; 1.1997x over previous
import jax
import jax.numpy as jnp
from jax.experimental import pallas as pl
from jax.experimental.pallas import tpu as pltpu


def _ln_mm1_body(x_ref, mu_ref, var_ref, g_ref, b_ref, w1_ref, b1_ref, o_ref):
    x = x_ref[0]
    xn = (x - mu_ref[0]) / jnp.sqrt(var_ref[0] + 1e-5) * g_ref[0] + b_ref[0]
    o_ref[0] = xn @ w1_ref[...] + b1_ref[0]


def _mm2_body(h_ref, gm_ref, w2_ref, b2_ref, o_ref):
    h = h_ref[0]
    n, c = h.shape
    x2 = jnp.concatenate(
        [h[:, : c // 2], jnp.broadcast_to(gm_ref[0], (n, c // 2))], axis=-1)
    o_ref[0] = x2 @ w2_ref[...] + b2_ref[0]


def _mm3_score_body(h2_ref, w3_ref, o_ref):
    logit = h2_ref[0] @ w3_ref[...]
    n = logit.shape[0]
    o_ref[0, 0] = jax.nn.sigmoid(logit).reshape(n)


def _bitonic_body(s_ref, o_ref):
    keys = jax.lax.bitcast_convert_type(s_ref[...], jnp.int32)
    b, n = keys.shape
    idx = jax.lax.broadcasted_iota(jnp.int32, (b, n), 1)
    lane = jax.lax.broadcasted_iota(jnp.int32, (1, n), 1)

    def exchange(keys, idx, j, k):
        km = jnp.concatenate([keys[:, j:], keys[:, :j]], axis=1)
        kp = jnp.concatenate([keys[:, n - j:], keys[:, :n - j]], axis=1)
        im = jnp.concatenate([idx[:, j:], idx[:, :j]], axis=1)
        ip = jnp.concatenate([idx[:, n - j:], idx[:, :n - j]], axis=1)
        is_low = (lane & j) == 0
        pk = jnp.where(is_low, km, kp)
        pi = jnp.where(is_low, im, ip)
        less = (keys > pk) | ((keys == pk) & (idx < pi))
        asc = (lane & k) == 0
        keep = less ^ (~is_low) ^ (~asc)
        return jnp.where(keep, keys, pk), jnp.where(keep, idx, pi)

    k = 2
    while k <= n:
        j = k // 2
        while j >= 1:
            keys, idx = exchange(keys, idx, j, k)
            j //= 2
        k *= 2
    o_ref[...] = idx


def _zmap2(b):
    return (0, 0)


def _bmap3(b):
    return (b, 0, 0)


def kernel(input_x, quality, ln_g, ln_b, w1, b1, w2, b2, w3):
    B, H, W, D = input_x.shape
    N = H * W
    C1 = w1.shape[1]
    C2 = w2.shape[1]
    f32 = jnp.float32

    mu = jnp.mean(input_x, axis=-1, keepdims=True)
    var = jnp.mean((input_x - mu) ** 2, axis=-1, keepdims=True)

    y1 = pl.pallas_call(
        _ln_mm1_body,
        grid=(B,),
        in_specs=[
            pl.BlockSpec((1, N, D), _bmap3),
            pl.BlockSpec((1, N, 1), _bmap3),
            pl.BlockSpec((1, N, 1), _bmap3),
            pl.BlockSpec((1, D), _zmap2),
            pl.BlockSpec((1, D), _zmap2),
            pl.BlockSpec((D, C1), _zmap2),
            pl.BlockSpec((1, C1), _zmap2),
        ],
        out_specs=pl.BlockSpec((1, N, C1), _bmap3),
        out_shape=jax.ShapeDtypeStruct((B, N, C1), f32),
    )(input_x.reshape(B, N, D), mu.reshape(B, N, 1), var.reshape(B, N, 1),
      ln_g.reshape(1, D), ln_b.reshape(1, D), w1, b1.reshape(1, C1))

    h = jax.nn.gelu(y1.reshape(B, H, W, C1), approximate=False)
    h = jax.lax.optimization_barrier(h)
    gmean = jnp.mean(h[:, :, :, C1 // 2:], axis=(1, 2), keepdims=True)

    y2 = pl.pallas_call(
        _mm2_body,
        grid=(B,),
        in_specs=[
            pl.BlockSpec((1, N, C1), _bmap3),
            pl.BlockSpec((1, 1, C1 // 2), _bmap3),
            pl.BlockSpec((C1, C2), _zmap2),
            pl.BlockSpec((1, C2), _zmap2),
        ],
        out_specs=pl.BlockSpec((1, N, C2), _bmap3),
        out_shape=jax.ShapeDtypeStruct((B, N, C2), f32),
    )(h.reshape(B, N, C1), gmean.reshape(B, 1, C1 // 2), w2,
      b2.reshape(1, C2))

    h2 = jax.nn.gelu(y2, approximate=False)

    scores = pl.pallas_call(
        _mm3_score_body,
        grid=(B,),
        in_specs=[
            pl.BlockSpec((1, N, C2), _bmap3),
            pl.BlockSpec((C2, 1), _zmap2),
        ],
        out_specs=pl.BlockSpec((1, 1, N), _bmap3),
        out_shape=jax.ShapeDtypeStruct((B, 1, N), f32),
    )(h2, w3)

    idx = pl.pallas_call(
        _bitonic_body,
        in_specs=[pl.BlockSpec((B, N), lambda: (0, 0))],
        out_specs=pl.BlockSpec((B, N), lambda: (0, 0)),
        out_shape=jax.ShapeDtypeStruct((B, N), jnp.int32),
    )(scores.reshape(B, N))
    log_base = 5.0
    quality_static = 4
    ratio = (log_base ** ((quality_static - 1) / 7.0) - 1.0) / (log_base - 1.0)
    num_keep = int(N * ratio)
    return (input_x, idx[:, :num_keep], idx[:, num_keep:])


# speedup vs baseline: 2.7032x; 1.0772x over previous
import jax
import jax.numpy as jnp
from jax.experimental import pallas as pl
from jax.experimental.pallas import tpu as pltpu


def _ln_mm1_body(x_ref, mu_ref, var_ref, g_ref, b_ref, w1_ref, b1_ref, o_ref,
                 xout_ref):
    x = x_ref[0]
    xn = (x - mu_ref[0]) / jnp.sqrt(var_ref[0] + 1e-5) * g_ref[0] + b_ref[0]
    o_ref[0] = xn @ w1_ref[...] + b1_ref[0]
    xout_ref[0] = x


def _mm2_body(h_ref, gm_ref, w2_ref, b2_ref, o_ref):
    h = h_ref[0]
    n, c = h.shape
    x2 = jnp.concatenate(
        [h[:, : c // 2], jnp.broadcast_to(gm_ref[0], (n, c // 2))], axis=-1)
    o_ref[0] = x2 @ w2_ref[...] + b2_ref[0]


def _mm3_score_body(h2_ref, w3_ref, o_ref):
    logit = h2_ref[0] @ w3_ref[...]
    n = logit.shape[0]
    o_ref[0, 0] = jax.nn.sigmoid(logit).reshape(n)


def _bitonic_body(s_ref, o_ref):
    keys = jax.lax.bitcast_convert_type(s_ref[...], jnp.int32)
    b, n = keys.shape
    idx = jax.lax.broadcasted_iota(jnp.int32, (b, n), 1)
    lane = jax.lax.broadcasted_iota(jnp.int32, (1, n), 1)

    def exchange(keys, idx, j, k):
        km = jnp.concatenate([keys[:, j:], keys[:, :j]], axis=1)
        kp = jnp.concatenate([keys[:, n - j:], keys[:, :n - j]], axis=1)
        im = jnp.concatenate([idx[:, j:], idx[:, :j]], axis=1)
        ip = jnp.concatenate([idx[:, n - j:], idx[:, :n - j]], axis=1)
        is_low = (lane & j) == 0
        pk = jnp.where(is_low, km, kp)
        pi = jnp.where(is_low, im, ip)
        less = (keys > pk) | ((keys == pk) & (idx < pi))
        asc = (lane & k) == 0
        keep = less ^ (~is_low) ^ (~asc)
        return jnp.where(keep, keys, pk), jnp.where(keep, idx, pi)

    k = 2
    while k <= n:
        j = k // 2
        while j >= 1:
            keys, idx = exchange(keys, idx, j, k)
            j //= 2
        k *= 2
    o_ref[...] = idx


def _zmap2(b):
    return (0, 0)


def _bmap3(b):
    return (b, 0, 0)


def kernel(input_x, quality, ln_g, ln_b, w1, b1, w2, b2, w3):
    B, H, W, D = input_x.shape
    N = H * W
    C1 = w1.shape[1]
    C2 = w2.shape[1]
    f32 = jnp.float32

    mu = jnp.mean(input_x, axis=-1, keepdims=True)
    var = jnp.mean((input_x - mu) ** 2, axis=-1, keepdims=True)

    y1, x_out = pl.pallas_call(
        _ln_mm1_body,
        grid=(B,),
        in_specs=[
            pl.BlockSpec((1, N, D), _bmap3),
            pl.BlockSpec((1, N, 1), _bmap3),
            pl.BlockSpec((1, N, 1), _bmap3),
            pl.BlockSpec((1, D), _zmap2),
            pl.BlockSpec((1, D), _zmap2),
            pl.BlockSpec((D, C1), _zmap2),
            pl.BlockSpec((1, C1), _zmap2),
        ],
        out_specs=[pl.BlockSpec((1, N, C1), _bmap3),
                   pl.BlockSpec((1, N, D), _bmap3)],
        out_shape=[jax.ShapeDtypeStruct((B, N, C1), f32),
                   jax.ShapeDtypeStruct((B, N, D), f32)],
    )(input_x.reshape(B, N, D), mu.reshape(B, N, 1), var.reshape(B, N, 1),
      ln_g.reshape(1, D), ln_b.reshape(1, D), w1, b1.reshape(1, C1))

    h = jax.nn.gelu(y1.reshape(B, H, W, C1), approximate=False)
    h = jax.lax.optimization_barrier(h)
    gmean = jnp.mean(h[:, :, :, C1 // 2:], axis=(1, 2), keepdims=True)

    y2 = pl.pallas_call(
        _mm2_body,
        grid=(B,),
        in_specs=[
            pl.BlockSpec((1, N, C1), _bmap3),
            pl.BlockSpec((1, 1, C1 // 2), _bmap3),
            pl.BlockSpec((C1, C2), _zmap2),
            pl.BlockSpec((1, C2), _zmap2),
        ],
        out_specs=pl.BlockSpec((1, N, C2), _bmap3),
        out_shape=jax.ShapeDtypeStruct((B, N, C2), f32),
    )(h.reshape(B, N, C1), gmean.reshape(B, 1, C1 // 2), w2,
      b2.reshape(1, C2))

    h2 = jax.nn.gelu(y2, approximate=False)

    scores = pl.pallas_call(
        _mm3_score_body,
        grid=(B,),
        in_specs=[
            pl.BlockSpec((1, N, C2), _bmap3),
            pl.BlockSpec((C2, 1), _zmap2),
        ],
        out_specs=pl.BlockSpec((1, 1, N), _bmap3),
        out_shape=jax.ShapeDtypeStruct((B, 1, N), f32),
    )(h2, w3)

    idx = pl.pallas_call(
        _bitonic_body,
        in_specs=[pl.BlockSpec((B, N), lambda: (0, 0))],
        out_specs=pl.BlockSpec((B, N), lambda: (0, 0)),
        out_shape=jax.ShapeDtypeStruct((B, N), jnp.int32),
    )(scores.reshape(B, N))
    log_base = 5.0
    quality_static = 4
    ratio = (log_base ** ((quality_static - 1) / 7.0) - 1.0) / (log_base - 1.0)
    num_keep = int(N * ratio)
    return (x_out.reshape(B, H, W, D), idx[:, :num_keep], idx[:, num_keep:])


# speedup vs baseline: 3.4662x; 1.2823x over previous
import jax
import jax.numpy as jnp
from jax.experimental import pallas as pl
from jax.experimental.pallas import tpu as pltpu


def _ln_mm1_body(x_ref, mu_ref, var_ref, g_ref, b_ref, w1_ref, b1_ref, o_ref,
                 xout_ref):
    x = x_ref[0]
    n = x.shape[0]
    mu = mu_ref[0].reshape(n, 1)
    var = var_ref[0].reshape(n, 1)
    xn = (x - mu) / jnp.sqrt(var + 1e-5) * g_ref[0] + b_ref[0]
    o_ref[0] = xn @ w1_ref[...] + b1_ref[0]
    xout_ref[0] = x


def _mm2_body(h_ref, gm_ref, w2_ref, b2_ref, o_ref):
    h = h_ref[0]
    n, c = h.shape
    x2 = jnp.concatenate(
        [h[:, : c // 2], jnp.broadcast_to(gm_ref[0], (n, c // 2))], axis=-1)
    o_ref[0] = (x2 @ w2_ref[...] + b2_ref[0]).T


def _mm3_score_body(h2t_ref, w3_ref, o_ref):
    logit = w3_ref[...] @ h2t_ref[0]
    o_ref[0] = jax.nn.sigmoid(logit)


def _bitonic_body(s_ref, o_ref):
    keys = jax.lax.bitcast_convert_type(s_ref[...], jnp.int32)
    b, n = keys.shape
    idx = jax.lax.broadcasted_iota(jnp.int32, (b, n), 1)
    lane = jax.lax.broadcasted_iota(jnp.int32, (1, n), 1)

    def exchange(keys, idx, j, k):
        km = jnp.concatenate([keys[:, j:], keys[:, :j]], axis=1)
        kp = jnp.concatenate([keys[:, n - j:], keys[:, :n - j]], axis=1)
        im = jnp.concatenate([idx[:, j:], idx[:, :j]], axis=1)
        ip = jnp.concatenate([idx[:, n - j:], idx[:, :n - j]], axis=1)
        is_low = (lane & j) == 0
        pk = jnp.where(is_low, km, kp)
        pi = jnp.where(is_low, im, ip)
        less = (keys > pk) | ((keys == pk) & (idx < pi))
        asc = (lane & k) == 0
        keep = less ^ (~is_low) ^ (~asc)
        return jnp.where(keep, keys, pk), jnp.where(keep, idx, pi)

    k = 2
    while k <= n:
        j = k // 2
        while j >= 1:
            keys, idx = exchange(keys, idx, j, k)
            j //= 2
        k *= 2
    o_ref[...] = idx


def _zmap2(b):
    return (0, 0)


def _bmap3(b):
    return (b, 0, 0)


def kernel(input_x, quality, ln_g, ln_b, w1, b1, w2, b2, w3):
    B, H, W, D = input_x.shape
    N = H * W
    C1 = w1.shape[1]
    C2 = w2.shape[1]
    f32 = jnp.float32

    mu = jnp.mean(input_x, axis=-1, keepdims=True)
    var = jnp.mean((input_x - mu) ** 2, axis=-1, keepdims=True)

    y1, x_out = pl.pallas_call(
        _ln_mm1_body,
        grid=(B,),
        in_specs=[
            pl.BlockSpec((1, N, D), _bmap3),
            pl.BlockSpec((1, 1, N), _bmap3),
            pl.BlockSpec((1, 1, N), _bmap3),
            pl.BlockSpec((1, D), _zmap2),
            pl.BlockSpec((1, D), _zmap2),
            pl.BlockSpec((D, C1), _zmap2),
            pl.BlockSpec((1, C1), _zmap2),
        ],
        out_specs=[pl.BlockSpec((1, N, C1), _bmap3),
                   pl.BlockSpec((1, N, D), _bmap3)],
        out_shape=[jax.ShapeDtypeStruct((B, N, C1), f32),
                   jax.ShapeDtypeStruct((B, N, D), f32)],
    )(input_x.reshape(B, N, D), mu.reshape(B, 1, N), var.reshape(B, 1, N),
      ln_g.reshape(1, D), ln_b.reshape(1, D), w1, b1.reshape(1, C1))

    h = jax.nn.gelu(y1.reshape(B, H, W, C1), approximate=False)
    h = jax.lax.optimization_barrier(h)
    gmean = jnp.mean(h[:, :, :, C1 // 2:], axis=(1, 2), keepdims=True)

    y2 = pl.pallas_call(
        _mm2_body,
        grid=(B,),
        in_specs=[
            pl.BlockSpec((1, N, C1), _bmap3),
            pl.BlockSpec((1, 1, C1 // 2), _bmap3),
            pl.BlockSpec((C1, C2), _zmap2),
            pl.BlockSpec((1, C2), _zmap2),
        ],
        out_specs=pl.BlockSpec((1, C2, N), _bmap3),
        out_shape=jax.ShapeDtypeStruct((B, C2, N), f32),
    )(h.reshape(B, N, C1), gmean.reshape(B, 1, C1 // 2), w2,
      b2.reshape(1, C2))

    h2t = jax.nn.gelu(y2, approximate=False)

    scores = pl.pallas_call(
        _mm3_score_body,
        grid=(B,),
        in_specs=[
            pl.BlockSpec((1, C2, N), _bmap3),
            pl.BlockSpec((1, C2), _zmap2),
        ],
        out_specs=pl.BlockSpec((1, 1, N), _bmap3),
        out_shape=jax.ShapeDtypeStruct((B, 1, N), f32),
    )(h2t, w3.reshape(1, C2))

    idx = pl.pallas_call(
        _bitonic_body,
        in_specs=[pl.BlockSpec((B, N), lambda: (0, 0))],
        out_specs=pl.BlockSpec((B, N), lambda: (0, 0)),
        out_shape=jax.ShapeDtypeStruct((B, N), jnp.int32),
    )(scores.reshape(B, N))
    log_base = 5.0
    quality_static = 4
    ratio = (log_base ** ((quality_static - 1) / 7.0) - 1.0) / (log_base - 1.0)
    num_keep = int(N * ratio)
    return (x_out.reshape(B, H, W, D), idx[:, :num_keep], idx[:, num_keep:])
